# baseline (speedup 1.0000x reference)
import jax
import jax.numpy as jnp
from jax.experimental import pallas as pl
from jax.experimental.pallas import tpu as pltpu

_HIDDEN = 768
_NUM_MOD = 64
_N_TOK = 2048
_TILE = 256
_NUM_TILES = _N_TOK // _TILE
_MAX_PAIRS = _NUM_TILES + _NUM_MOD - 1


def _gmm_kernel(meta_ref, x_ref, w_ref, nw_ref, out_ref):
    j = pl.program_id(0)
    tile = meta_ref[0, j]
    row_lo = meta_ref[2, j]
    row_hi = meta_ref[3, j]
    prev_tile = meta_ref[0, jnp.maximum(j - 1, 0)]
    first = jnp.logical_or(j == 0, tile != prev_tile)

    normed = (x_ref[...] * (nw_ref[0] + 1.0)).astype(jnp.bfloat16)
    y = jax.lax.dot_general(
        normed,
        w_ref[0],
        dimension_numbers=(((1,), (1,)), ((), ())),
        preferred_element_type=jnp.float32,
    )

    rows = jax.lax.broadcasted_iota(jnp.int32, (_TILE, 1), 0)
    mask = jnp.logical_and(rows >= row_lo, rows < row_hi)

    @pl.when(first)
    def _():
        out_ref[...] = jnp.where(mask, y, 0.0)

    @pl.when(jnp.logical_not(first))
    def _():
        out_ref[...] = jnp.where(mask, y, out_ref[...])


def _build_meta(mm):
    e_ids = jnp.arange(_NUM_MOD, dtype=jnp.int32)
    ends = jnp.sum(mm[None, :] <= e_ids[:, None], axis=1).astype(jnp.int32)
    starts = jnp.sum(mm[None, :] < e_ids[:, None], axis=1).astype(jnp.int32)
    first_e = mm[:: _TILE]
    last_e = mm[_TILE - 1 :: _TILE]
    off = jnp.cumsum(last_e - first_e + 1).astype(jnp.int32)
    j = jnp.arange(_MAX_PAIRS, dtype=jnp.int32)
    t_j = jnp.minimum(
        jnp.sum(off[None, :] <= j[:, None], axis=1).astype(jnp.int32),
        _NUM_TILES - 1,
    )
    prev_off = jnp.where(t_j > 0, off[jnp.maximum(t_j - 1, 0)], 0).astype(jnp.int32)
    e_j = jnp.clip(first_e[t_j] + (j - prev_off), first_e[t_j], last_e[t_j])
    row_lo = jnp.clip(starts[e_j] - t_j * _TILE, 0, _TILE)
    row_hi = jnp.clip(ends[e_j] - t_j * _TILE, 0, _TILE)
    return jnp.stack([t_j, e_j, row_lo, row_hi], axis=0)


def kernel(x, modality_mapping, W, norm_w):
    mm = modality_mapping.astype(jnp.int32)
    meta = _build_meta(mm)

    grid_spec = pltpu.PrefetchScalarGridSpec(
        num_scalar_prefetch=1,
        grid=(_MAX_PAIRS,),
        in_specs=[
            pl.BlockSpec((_TILE, _HIDDEN), lambda j, m: (m[0, j], 0)),
            pl.BlockSpec((1, _HIDDEN, _HIDDEN), lambda j, m: (m[1, j], 0, 0)),
            pl.BlockSpec((1, 1, _HIDDEN), lambda j, m: (m[1, j], 0, 0)),
        ],
        out_specs=pl.BlockSpec((_TILE, _HIDDEN), lambda j, m: (m[0, j], 0)),
    )
    return pl.pallas_call(
        _gmm_kernel,
        grid_spec=grid_spec,
        out_shape=jax.ShapeDtypeStruct((_N_TOK, _HIDDEN), jnp.float32),
        compiler_params=pltpu.CompilerParams(
            dimension_semantics=("arbitrary",)
        ),
    )(meta, x, W, norm_w.reshape(_NUM_MOD, 1, _HIDDEN))


# speedup vs baseline: 1.0014x; 1.0014x over previous
import jax
import jax.numpy as jnp
from jax.experimental import pallas as pl
from jax.experimental.pallas import tpu as pltpu

_HIDDEN = 768
_NUM_MOD = 64
_N_TOK = 2048
_TILE = 64
_NUM_TILES = _N_TOK // _TILE
_MAX_PAIRS = _NUM_TILES + _NUM_MOD - 1


def _gmm_kernel(meta_ref, x_ref, w_ref, nw_ref, out_ref):
    j = pl.program_id(0)
    tile = meta_ref[0, j]
    row_lo = meta_ref[2, j]
    row_hi = meta_ref[3, j]
    prev_tile = meta_ref[0, jnp.maximum(j - 1, 0)]
    first = jnp.logical_or(j == 0, tile != prev_tile)

    normed = (x_ref[...] * (nw_ref[0] + 1.0)).astype(jnp.bfloat16)
    y = jax.lax.dot_general(
        normed,
        w_ref[0],
        dimension_numbers=(((1,), (1,)), ((), ())),
        preferred_element_type=jnp.float32,
    )

    rows = jax.lax.broadcasted_iota(jnp.int32, (_TILE, 1), 0)
    mask = jnp.logical_and(rows >= row_lo, rows < row_hi)

    @pl.when(first)
    def _():
        out_ref[...] = jnp.where(mask, y, 0.0)

    @pl.when(jnp.logical_not(first))
    def _():
        out_ref[...] = jnp.where(mask, y, out_ref[...])


def _build_meta(mm):
    e_ids = jnp.arange(_NUM_MOD, dtype=jnp.int32)
    ends = jnp.sum(mm[None, :] <= e_ids[:, None], axis=1).astype(jnp.int32)
    starts = jnp.sum(mm[None, :] < e_ids[:, None], axis=1).astype(jnp.int32)
    first_e = mm[:: _TILE]
    last_e = mm[_TILE - 1 :: _TILE]
    off = jnp.cumsum(last_e - first_e + 1).astype(jnp.int32)
    j = jnp.arange(_MAX_PAIRS, dtype=jnp.int32)
    t_j = jnp.minimum(
        jnp.sum(off[None, :] <= j[:, None], axis=1).astype(jnp.int32),
        _NUM_TILES - 1,
    )
    prev_off = jnp.where(t_j > 0, off[jnp.maximum(t_j - 1, 0)], 0).astype(jnp.int32)
    e_j = jnp.clip(first_e[t_j] + (j - prev_off), first_e[t_j], last_e[t_j])
    row_lo = jnp.clip(starts[e_j] - t_j * _TILE, 0, _TILE)
    row_hi = jnp.clip(ends[e_j] - t_j * _TILE, 0, _TILE)
    return jnp.stack([t_j, e_j, row_lo, row_hi], axis=0)


def kernel(x, modality_mapping, W, norm_w):
    mm = modality_mapping.astype(jnp.int32)
    meta = _build_meta(mm)

    grid_spec = pltpu.PrefetchScalarGridSpec(
        num_scalar_prefetch=1,
        grid=(_MAX_PAIRS,),
        in_specs=[
            pl.BlockSpec((_TILE, _HIDDEN), lambda j, m: (m[0, j], 0)),
            pl.BlockSpec((1, _HIDDEN, _HIDDEN), lambda j, m: (m[1, j], 0, 0)),
            pl.BlockSpec((1, 1, _HIDDEN), lambda j, m: (m[1, j], 0, 0)),
        ],
        out_specs=pl.BlockSpec((_TILE, _HIDDEN), lambda j, m: (m[0, j], 0)),
    )
    return pl.pallas_call(
        _gmm_kernel,
        grid_spec=grid_spec,
        out_shape=jax.ShapeDtypeStruct((_N_TOK, _HIDDEN), jnp.float32),
        compiler_params=pltpu.CompilerParams(
            dimension_semantics=("arbitrary",)
        ),
    )(meta, x, W, norm_w.reshape(_NUM_MOD, 1, _HIDDEN))


# speedup vs baseline: 1.2695x; 1.2677x over previous
import jax
import jax.numpy as jnp
from jax.experimental import pallas as pl
from jax.experimental.pallas import tpu as pltpu

_HIDDEN = 768
_NUM_MOD = 64
_N_TOK = 2048
_TILE = 128


def _gmm_kernel(se_ref, x_ref, w_ref, nw_ref, out_ref):
    e = pl.program_id(0)
    seg_lo = se_ref[0, e]
    seg_hi = se_ref[1, e]
    t_lo = seg_lo // _TILE
    t_hi = (seg_hi + _TILE - 1) // _TILE

    scale = nw_ref[0] + 1.0

    def body(t, _):
        r0 = t * _TILE
        xs = x_ref[pl.ds(r0, _TILE), :]
        normed = (xs * scale).astype(jnp.bfloat16)
        y = jax.lax.dot_general(
            normed,
            w_ref[0],
            dimension_numbers=(((1,), (1,)), ((), ())),
            preferred_element_type=jnp.float32,
        )
        rows = r0 + jax.lax.broadcasted_iota(jnp.int32, (_TILE, 1), 0)
        mask = jnp.logical_and(rows >= seg_lo, rows < seg_hi)
        out_ref[pl.ds(r0, _TILE), :] = jnp.where(
            mask, y, out_ref[pl.ds(r0, _TILE), :]
        )
        return 0

    jax.lax.fori_loop(t_lo, t_hi, body, 0)


def kernel(x, modality_mapping, W, norm_w):
    mm = modality_mapping.astype(jnp.int32)
    e_ids = jnp.arange(_NUM_MOD, dtype=jnp.int32)
    ends = jnp.sum(mm[None, :] <= e_ids[:, None], axis=1).astype(jnp.int32)
    starts = jnp.sum(mm[None, :] < e_ids[:, None], axis=1).astype(jnp.int32)
    se = jnp.stack([starts, ends], axis=0)

    grid_spec = pltpu.PrefetchScalarGridSpec(
        num_scalar_prefetch=1,
        grid=(_NUM_MOD,),
        in_specs=[
            pl.BlockSpec((_N_TOK, _HIDDEN), lambda e, s: (0, 0)),
            pl.BlockSpec((1, _HIDDEN, _HIDDEN), lambda e, s: (e, 0, 0)),
            pl.BlockSpec((1, 1, _HIDDEN), lambda e, s: (e, 0, 0)),
        ],
        out_specs=pl.BlockSpec((_N_TOK, _HIDDEN), lambda e, s: (0, 0)),
    )
    return pl.pallas_call(
        _gmm_kernel,
        grid_spec=grid_spec,
        out_shape=jax.ShapeDtypeStruct((_N_TOK, _HIDDEN), jnp.float32),
        compiler_params=pltpu.CompilerParams(
            dimension_semantics=("arbitrary",)
        ),
    )(se, x, W, norm_w.reshape(_NUM_MOD, 1, _HIDDEN))


# speedup vs baseline: 1.6760x; 1.3202x over previous
import jax
import jax.numpy as jnp
from jax.experimental import pallas as pl
from jax.experimental.pallas import tpu as pltpu

_HIDDEN = 768
_NUM_MOD = 64
_N_TOK = 2048
_TILE = 128
_NBUF = 3


def _gmm_kernel(se_ref, x_ref, w_hbm, nw_ref, out_ref, wbuf, sems):
    def start_copy(e, slot):
        pltpu.make_async_copy(w_hbm.at[e], wbuf.at[slot], sems.at[slot]).start()

    for k in range(_NBUF - 1):
        start_copy(k, k)

    def expert_step(e, _):
        nxt = e + _NBUF - 1

        @pl.when(nxt < _NUM_MOD)
        def _():
            start_copy(nxt, jax.lax.rem(nxt, _NBUF))

        slot = jax.lax.rem(e, _NBUF)
        pltpu.make_async_copy(
            w_hbm.at[e], wbuf.at[slot], sems.at[slot]
        ).wait()

        seg_lo = se_ref[0, e]
        seg_hi = se_ref[1, e]
        scale = nw_ref[e] + 1.0

        def tile_step(t, _):
            r0 = t * _TILE
            xs = x_ref[pl.ds(r0, _TILE), :]
            normed = (xs * scale).astype(jnp.bfloat16)
            y = jax.lax.dot_general(
                normed,
                wbuf[slot],
                dimension_numbers=(((1,), (1,)), ((), ())),
                preferred_element_type=jnp.float32,
            )
            rows = r0 + jax.lax.broadcasted_iota(jnp.int32, (_TILE, 1), 0)
            mask = jnp.logical_and(rows >= seg_lo, rows < seg_hi)
            out_ref[pl.ds(r0, _TILE), :] = jnp.where(
                mask, y, out_ref[pl.ds(r0, _TILE), :]
            )
            return 0

        jax.lax.fori_loop(
            seg_lo // _TILE, (seg_hi + _TILE - 1) // _TILE, tile_step, 0
        )
        return 0

    jax.lax.fori_loop(0, _NUM_MOD, expert_step, 0)


def kernel(x, modality_mapping, W, norm_w):
    mm = modality_mapping.astype(jnp.int32)
    e_ids = jnp.arange(_NUM_MOD, dtype=jnp.int32)
    ends = jnp.sum(mm[None, :] <= e_ids[:, None], axis=1).astype(jnp.int32)
    starts = jnp.sum(mm[None, :] < e_ids[:, None], axis=1).astype(jnp.int32)
    se = jnp.stack([starts, ends], axis=0)

    grid_spec = pltpu.PrefetchScalarGridSpec(
        num_scalar_prefetch=1,
        grid=(1,),
        in_specs=[
            pl.BlockSpec((_N_TOK, _HIDDEN), lambda g, s: (0, 0)),
            pl.BlockSpec(memory_space=pl.ANY),
            pl.BlockSpec((_NUM_MOD, _HIDDEN), lambda g, s: (0, 0)),
        ],
        out_specs=pl.BlockSpec((_N_TOK, _HIDDEN), lambda g, s: (0, 0)),
        scratch_shapes=[
            pltpu.VMEM((_NBUF, _HIDDEN, _HIDDEN), jnp.bfloat16),
            pltpu.SemaphoreType.DMA((_NBUF,)),
        ],
    )
    return pl.pallas_call(
        _gmm_kernel,
        grid_spec=grid_spec,
        out_shape=jax.ShapeDtypeStruct((_N_TOK, _HIDDEN), jnp.float32),
        compiler_params=pltpu.CompilerParams(
            dimension_semantics=("arbitrary",)
        ),
    )(se, x, W, norm_w)


# speedup vs baseline: 1.7109x; 1.0208x over previous
import jax
import jax.numpy as jnp
from jax.experimental import pallas as pl
from jax.experimental.pallas import tpu as pltpu

_HIDDEN = 768
_NUM_MOD = 64
_N_TOK = 2048
_TILE = 128
_NBUF = 4


def _gmm_kernel(se_ref, x_ref, w_hbm, nw_ref, out_ref, wbuf, sems):
    def start_copy(e, slot):
        pltpu.make_async_copy(w_hbm.at[e], wbuf.at[slot], sems.at[slot]).start()

    for k in range(_NBUF - 1):
        start_copy(k, k)

    def expert_step(e, _):
        nxt = e + _NBUF - 1

        @pl.when(nxt < _NUM_MOD)
        def _():
            start_copy(nxt, jax.lax.rem(nxt, _NBUF))

        slot = jax.lax.rem(e, _NBUF)
        pltpu.make_async_copy(
            w_hbm.at[e], wbuf.at[slot], sems.at[slot]
        ).wait()

        seg_lo = se_ref[0, e]
        seg_hi = se_ref[1, e]
        scale = nw_ref[e] + 1.0

        def tile_step(t, _):
            r0 = t * _TILE
            xs = x_ref[pl.ds(r0, _TILE), :]
            normed = (xs * scale).astype(jnp.bfloat16)
            y = jax.lax.dot_general(
                normed,
                wbuf[slot],
                dimension_numbers=(((1,), (1,)), ((), ())),
                preferred_element_type=jnp.float32,
            )
            rows = r0 + jax.lax.broadcasted_iota(jnp.int32, (_TILE, 1), 0)
            mask = jnp.logical_and(rows >= seg_lo, rows < seg_hi)
            out_ref[pl.ds(r0, _TILE), :] = jnp.where(
                mask, y, out_ref[pl.ds(r0, _TILE), :]
            )
            return 0

        jax.lax.fori_loop(
            seg_lo // _TILE, (seg_hi + _TILE - 1) // _TILE, tile_step, 0
        )
        return 0

    jax.lax.fori_loop(0, _NUM_MOD, expert_step, 0)


def kernel(x, modality_mapping, W, norm_w):
    mm = modality_mapping.astype(jnp.int32)
    e_ids = jnp.arange(_NUM_MOD, dtype=jnp.int32)
    ends = jnp.sum(mm[None, :] <= e_ids[:, None], axis=1).astype(jnp.int32)
    starts = jnp.sum(mm[None, :] < e_ids[:, None], axis=1).astype(jnp.int32)
    se = jnp.stack([starts, ends], axis=0)

    grid_spec = pltpu.PrefetchScalarGridSpec(
        num_scalar_prefetch=1,
        grid=(1,),
        in_specs=[
            pl.BlockSpec((_N_TOK, _HIDDEN), lambda g, s: (0, 0)),
            pl.BlockSpec(memory_space=pl.ANY),
            pl.BlockSpec((_NUM_MOD, _HIDDEN), lambda g, s: (0, 0)),
        ],
        out_specs=pl.BlockSpec((_N_TOK, _HIDDEN), lambda g, s: (0, 0)),
        scratch_shapes=[
            pltpu.VMEM((_NBUF, _HIDDEN, _HIDDEN), jnp.bfloat16),
            pltpu.SemaphoreType.DMA((_NBUF,)),
        ],
    )
    return pl.pallas_call(
        _gmm_kernel,
        grid_spec=grid_spec,
        out_shape=jax.ShapeDtypeStruct((_N_TOK, _HIDDEN), jnp.float32),
        compiler_params=pltpu.CompilerParams(
            dimension_semantics=("arbitrary",)
        ),
    )(se, x, W, norm_w)


# speedup vs baseline: 1.7251x; 1.0083x over previous
import jax
import jax.numpy as jnp
from jax.experimental import pallas as pl
from jax.experimental.pallas import tpu as pltpu

_HIDDEN = 768
_NUM_MOD = 64
_N_TOK = 2048
_TILE = 128
_NBUF = 4


def _gmm_kernel(se_ref, x_ref, w_hbm, nw_ref, out_ref, wbuf, sems):
    def start_copy(e, slot):
        pltpu.make_async_copy(w_hbm.at[e], wbuf.at[slot], sems.at[slot]).start()

    for k in range(_NBUF - 1):
        start_copy(k, k)

    def process_expert(e):
        nxt = e + _NBUF - 1

        @pl.when(nxt < _NUM_MOD)
        def _():
            start_copy(nxt, jax.lax.rem(nxt, _NBUF))

        slot = jax.lax.rem(e, _NBUF)
        pltpu.make_async_copy(
            w_hbm.at[e], wbuf.at[slot], sems.at[slot]
        ).wait()

        seg_lo = se_ref[0, e]
        seg_hi = se_ref[1, e]
        scale = nw_ref[e] + 1.0

        def tile_step(t, _):
            r0 = t * _TILE
            xs = x_ref[pl.ds(r0, _TILE), :]
            normed = (xs * scale).astype(jnp.bfloat16)
            y = jax.lax.dot_general(
                normed,
                wbuf[slot],
                dimension_numbers=(((1,), (1,)), ((), ())),
                preferred_element_type=jnp.float32,
            )
            rows = r0 + jax.lax.broadcasted_iota(jnp.int32, (_TILE, 1), 0)
            mask = jnp.logical_and(rows >= seg_lo, rows < seg_hi)
            out_ref[pl.ds(r0, _TILE), :] = jnp.where(
                mask, y, out_ref[pl.ds(r0, _TILE), :]
            )
            return 0

        jax.lax.fori_loop(
            seg_lo // _TILE, (seg_hi + _TILE - 1) // _TILE, tile_step, 0
        )

    def expert_pair_step(i, _):
        process_expert(2 * i)
        process_expert(2 * i + 1)
        return 0

    jax.lax.fori_loop(0, _NUM_MOD // 2, expert_pair_step, 0)


def kernel(x, modality_mapping, W, norm_w):
    mm = modality_mapping.astype(jnp.int32)
    e_ids = jnp.arange(_NUM_MOD, dtype=jnp.int32)
    ends = jnp.sum(mm[None, :] <= e_ids[:, None], axis=1).astype(jnp.int32)
    starts = jnp.sum(mm[None, :] < e_ids[:, None], axis=1).astype(jnp.int32)
    se = jnp.stack([starts, ends], axis=0)

    grid_spec = pltpu.PrefetchScalarGridSpec(
        num_scalar_prefetch=1,
        grid=(1,),
        in_specs=[
            pl.BlockSpec((_N_TOK, _HIDDEN), lambda g, s: (0, 0)),
            pl.BlockSpec(memory_space=pl.ANY),
            pl.BlockSpec((_NUM_MOD, _HIDDEN), lambda g, s: (0, 0)),
        ],
        out_specs=pl.BlockSpec((_N_TOK, _HIDDEN), lambda g, s: (0, 0)),
        scratch_shapes=[
            pltpu.VMEM((_NBUF, _HIDDEN, _HIDDEN), jnp.bfloat16),
            pltpu.SemaphoreType.DMA((_NBUF,)),
        ],
    )
    return pl.pallas_call(
        _gmm_kernel,
        grid_spec=grid_spec,
        out_shape=jax.ShapeDtypeStruct((_N_TOK, _HIDDEN), jnp.float32),
        compiler_params=pltpu.CompilerParams(
            dimension_semantics=("arbitrary",)
        ),
    )(se, x, W, norm_w)
